# Initial kernel scaffold; baseline (speedup 1.0000x reference)
import jax
import jax.numpy as jnp
from jax.experimental import pallas as pl


def kernel(x, edge_index, W1, b1, W2, b2, W3, b3, Wfc, bfc):
    raise NotImplementedError("write your pallas kernel here")


# speedup vs baseline: 10.6977x; 10.6977x over previous
import functools

import jax
import jax.numpy as jnp
from jax import lax
from jax.experimental import pallas as pl
from jax.experimental.pallas import tpu as pltpu
from jax.experimental.pallas import tpu_sc as plsc

NC = 2
NS = 16
NW = NC * NS

def _make_deg_kernel(np_, f, e, ch):
    epw = e // NW
    nch = epw // ch
    rpt = np_ // NS
    lanes = 16
    mesh = plsc.VectorSubcoreMesh(core_axis_name="c", subcore_axis_name="s")

    @functools.partial(
        pl.kernel,
        mesh=mesh,
        out_type=jax.ShapeDtypeStruct((NC, np_, f), jnp.float32),
        scratch_types=[
            pltpu.VMEM((ch,), jnp.int32),
            pltpu.VMEM((ch, f), jnp.float32),
            pltpu.VMEM_SHARED((np_, f), jnp.float32),
        ],
    )
    def deg_kernel(dst_hbm, z_hbm, out_hbm, didx, ones_v, acc):
        c = lax.axis_index("c")
        s = lax.axis_index("s")
        wid = s * NC + c
        r0 = s * rpt

        def fill(j, carry):
            r = j // (f // lanes)
            col = (j % (f // lanes)) * lanes
            ones_v[r, pl.ds(col, lanes)] = jnp.full((lanes,), 1.0, jnp.float32)
            return carry

        lax.fori_loop(0, ch * (f // lanes), fill, 0)
        pltpu.sync_copy(z_hbm.at[pl.ds(r0, rpt)], acc.at[pl.ds(r0, rpt)])
        plsc.subcore_barrier()
        base = wid * epw

        def body(j, carry):
            off = pl.multiple_of(base + j * ch, 8)
            pltpu.sync_copy(dst_hbm.at[pl.ds(off, ch)], didx)
            pltpu.sync_copy(ones_v, acc.at[didx], add=True)
            return carry

        lax.fori_loop(0, nch, body, 0)
        plsc.subcore_barrier()
        pltpu.sync_copy(acc.at[pl.ds(r0, rpt)], out_hbm.at[c, pl.ds(r0, rpt)])

    return deg_kernel


def _make_agg_kernel(np_, f, e, ch):
    epw = e // NW
    nch = epw // ch
    rpt = np_ // NS
    mesh = plsc.VectorSubcoreMesh(core_axis_name="c", subcore_axis_name="s")

    @functools.partial(
        pl.kernel,
        mesh=mesh,
        out_type=jax.ShapeDtypeStruct((NC, np_, f), jnp.float32),
        scratch_types=[
            pltpu.VMEM((ch,), jnp.int32),
            pltpu.VMEM((ch,), jnp.int32),
            pltpu.VMEM((ch, f), jnp.float32),
            pltpu.VMEM_SHARED((np_, f), jnp.float32),
            pltpu.SemaphoreType.DMA,
        ],
    )
    def agg_kernel(src_hbm, dst_hbm, g_hbm, z_hbm, out_hbm, sidx, didx, rows, acc, sem):
        c = lax.axis_index("c")
        s = lax.axis_index("s")
        wid = s * NC + c
        r0 = s * rpt
        pltpu.sync_copy(z_hbm.at[pl.ds(r0, rpt)], acc.at[pl.ds(r0, rpt)])
        plsc.subcore_barrier()
        base = wid * epw

        def body(j, carry):
            off = pl.multiple_of(base + j * ch, 8)
            pltpu.sync_copy(src_hbm.at[pl.ds(off, ch)], sidx)
            pltpu.sync_copy(dst_hbm.at[pl.ds(off, ch)], didx)
            pltpu.async_copy(g_hbm.at[sidx], rows, sem).wait()
            pltpu.sync_copy(rows, acc.at[didx], add=True)
            return carry

        lax.fori_loop(0, nch, body, 0)
        plsc.subcore_barrier()
        pltpu.sync_copy(acc.at[pl.ds(r0, rpt)], out_hbm.at[c, pl.ds(r0, rpt)])

    return agg_kernel


def _dinv_from_degp(degp):
    deg = degp[0, :, 0:1] + degp[1, :, 0:1] + 1.0
    return lax.rsqrt(deg)


def _tca_body(x_ref, degp_ref, w_ref, out_ref):
    dinv = _dinv_from_degp(degp_ref[...])
    out_ref[...] = dinv * jnp.dot(
        x_ref[...], w_ref[...], preferred_element_type=jnp.float32
    )


def _tcb_body(sp_ref, g_ref, degp_ref, b_ref, w_ref, out_ref):
    dinv = _dinv_from_degp(degp_ref[...])
    g = g_ref[...]
    pre = dinv * (sp_ref[0] + sp_ref[1] + g) + b_ref[...]
    a = jnp.where(pre > 0, pre, jnp.exp(pre) - 1.0)
    out_ref[...] = dinv * jnp.dot(
        a, w_ref[...], preferred_element_type=jnp.float32
    )


def _tcc_body(sp_ref, g_ref, degp_ref, b_ref, wfc_ref, bfc_ref, out_ref):
    dinv = _dinv_from_degp(degp_ref[...])
    g = g_ref[...]
    pre = dinv * (sp_ref[0] + sp_ref[1] + g) + b_ref[...]
    a = jnp.where(pre > 0, pre, jnp.exp(pre) - 1.0)
    logits = jnp.dot(a, wfc_ref[...], preferred_element_type=jnp.float32)
    logits = logits + bfc_ref[...]
    m = jnp.max(logits, axis=-1, keepdims=True)
    lse = jnp.log(jnp.sum(jnp.exp(logits - m), axis=-1, keepdims=True))
    out_ref[...] = logits - m - lse


def kernel(x, edge_index, W1, b1, W2, b2, W3, b3, Wfc, bfc):
    n, f_in = x.shape
    h = W1.shape[1]
    c_out = Wfc.shape[1]
    e = edge_index.shape[1]
    ch = 80

    src = edge_index[0]
    dst = edge_index[1]
    np_ = ((n + NS * 8 - 1) // (NS * 8)) * (NS * 8)
    zeros_h = jnp.zeros((np_, h), jnp.float32)

    deg_k = _make_deg_kernel(np_, h, e, ch)
    agg_k = _make_agg_kernel(np_, h, e, ch)

    rb = 1000
    grid = (n // rb,)
    full = lambda shape: pl.BlockSpec(shape, lambda i: (0,) * len(shape))
    rows128 = pl.BlockSpec((rb, h), lambda i: (i, 0))
    degp_spec = pl.BlockSpec((NC, rb, h), lambda i: (0, i, 0))
    sp_spec = pl.BlockSpec((NC, rb, h), lambda i: (0, i, 0))

    degp = deg_k(dst, zeros_h)

    tca = pl.pallas_call(
        _tca_body,
        grid=grid,
        in_specs=[
            pl.BlockSpec((rb, f_in), lambda i: (i, 0)),
            degp_spec,
            full((f_in, h)),
        ],
        out_specs=rows128,
        out_shape=jax.ShapeDtypeStruct((n, h), jnp.float32),
    )
    g1 = tca(x, degp, W1)

    tcb = pl.pallas_call(
        _tcb_body,
        grid=grid,
        in_specs=[sp_spec, rows128, degp_spec, full((1, h)), full((h, h))],
        out_specs=rows128,
        out_shape=jax.ShapeDtypeStruct((n, h), jnp.float32),
    )
    tcc = pl.pallas_call(
        _tcc_body,
        grid=grid,
        in_specs=[
            sp_spec,
            rows128,
            degp_spec,
            full((1, h)),
            full((h, c_out)),
            full((1, c_out)),
        ],
        out_specs=pl.BlockSpec((rb, c_out), lambda i: (i, 0)),
        out_shape=jax.ShapeDtypeStruct((n, c_out), jnp.float32),
    )

    sp1 = agg_k(src, dst, g1, zeros_h)
    g2 = tcb(sp1, g1, degp, b1.reshape(1, h), W2)
    sp2 = agg_k(src, dst, g2, zeros_h)
    g3 = tcb(sp2, g2, degp, b2.reshape(1, h), W3)
    sp3 = agg_k(src, dst, g3, zeros_h)
    return tcc(sp3, g3, degp, b3.reshape(1, h), Wfc, bfc.reshape(1, c_out))


# speedup vs baseline: 21.6013x; 2.0192x over previous
import functools

import jax
import jax.numpy as jnp
from jax import lax
from jax.experimental import pallas as pl
from jax.experimental.pallas import tpu as pltpu
from jax.experimental.pallas import tpu_sc as plsc

NC = 2
NS = 16
NW = NC * NS

def _make_deg_kernel(np_, f, e, ch):
    epw = e // NW
    nch = epw // ch
    rpt = np_ // NS
    lanes = 16
    mesh = plsc.VectorSubcoreMesh(core_axis_name="c", subcore_axis_name="s")

    @functools.partial(
        pl.kernel,
        mesh=mesh,
        out_type=jax.ShapeDtypeStruct((NC, np_, f), jnp.float32),
        scratch_types=[
            pltpu.VMEM((ch,), jnp.int32),
            pltpu.VMEM((ch, f), jnp.float32),
            pltpu.VMEM_SHARED((np_, f), jnp.float32),
        ],
    )
    def deg_kernel(dst_hbm, z_hbm, out_hbm, didx, ones_v, acc):
        c = lax.axis_index("c")
        s = lax.axis_index("s")
        wid = s * NC + c
        r0 = s * rpt

        def fill(j, carry):
            r = j // (f // lanes)
            col = (j % (f // lanes)) * lanes
            ones_v[r, pl.ds(col, lanes)] = jnp.full((lanes,), 1.0, jnp.float32)
            return carry

        lax.fori_loop(0, ch * (f // lanes), fill, 0)
        pltpu.sync_copy(z_hbm.at[pl.ds(r0, rpt)], acc.at[pl.ds(r0, rpt)])
        plsc.subcore_barrier()
        base = wid * epw

        def body(j, carry):
            off = pl.multiple_of(base + j * ch, 8)
            pltpu.sync_copy(dst_hbm.at[pl.ds(off, ch)], didx)
            pltpu.sync_copy(ones_v, acc.at[didx], add=True)
            return carry

        lax.fori_loop(0, nch, body, 0)
        plsc.subcore_barrier()
        pltpu.sync_copy(acc.at[pl.ds(r0, rpt)], out_hbm.at[c, pl.ds(r0, rpt)])

    return deg_kernel


def _make_agg_kernel(np_, f, e, ch):
    epw = e // NW
    nch = epw // ch
    rpt = np_ // NS
    mesh = plsc.VectorSubcoreMesh(core_axis_name="c", subcore_axis_name="s")

    nb = 1
    nblk = nch // nb
    assert nch % nb == 0 and nblk >= 3 and nblk % 2 == 1

    @functools.partial(
        pl.kernel,
        mesh=mesh,
        out_type=jax.ShapeDtypeStruct((NC, np_, f), jnp.float32),
        scratch_types=[
            pltpu.VMEM((epw,), jnp.int32),
            pltpu.VMEM((nch, ch), jnp.int32),
            pltpu.VMEM((2, nb, ch, f), jnp.float32),
            pltpu.VMEM_SHARED((np_, f), jnp.float32),
            pltpu.SemaphoreType.DMA,
            pltpu.SemaphoreType.DMA,
            pltpu.SemaphoreType.DMA,
            pltpu.SemaphoreType.DMA,
            pltpu.SemaphoreType.DMA,
            pltpu.SemaphoreType.DMA,
        ],
    )
    def agg_kernel(src_hbm, dst_hbm, g_hbm, z_hbm, out_hbm, sidx_all, didx_all,
                   rows, acc, si0, si1, sg0, sg1, ss0, ss1):
        c = lax.axis_index("c")
        s = lax.axis_index("s")
        wid = s * NC + c
        r0 = s * rpt
        base = wid * epw
        sem_i = (si0, si1)
        sem_g = (sg0, sg1)
        sem_s = (ss0, ss1)

        sidx_cp = pltpu.async_copy(
            src_hbm.at[pl.ds(pl.multiple_of(base, 8), epw)], sidx_all, si0)
        pltpu.sync_copy(z_hbm.at[pl.ds(r0, rpt)], acc.at[pl.ds(r0, rpt)])
        plsc.subcore_barrier()
        sidx_cp.wait()

        def fire(t, st, drain_s):
            if drain_s:
                for b in range(nb):
                    pltpu.make_async_copy(
                        g_hbm.at[pl.ds(0, ch)], rows.at[st, b], sem_s[st]).wait()
            for b in range(nb):
                j = t * nb + b
                off = pl.multiple_of(base + j * ch, 8)
                pltpu.async_copy(dst_hbm.at[pl.ds(off, ch)], didx_all.at[j],
                                 sem_i[st])
                goff = pl.multiple_of(j * ch, 8)
                pltpu.async_copy(g_hbm.at[sidx_all.at[pl.ds(goff, ch)]],
                                 rows.at[st, b], sem_g[st])

        def drain(t, st):
            for b in range(nb):
                pltpu.make_async_copy(
                    g_hbm.at[pl.ds(0, ch)], rows.at[st, b], sem_g[st]).wait()
            for b in range(nb):
                j = t * nb + b
                pltpu.make_async_copy(
                    dst_hbm.at[pl.ds(0, ch)], didx_all.at[j], sem_i[st]).wait()
                pltpu.async_copy(rows.at[st, b], acc.at[didx_all.at[j]],
                                 sem_s[st], add=True)

        fire(0, 0, False)
        fire(1, 1, False)

        def body(i, carry):
            t = 2 * i
            drain(t, 0)
            fire(t + 2, 0, True)
            drain(t + 1, 1)
            fire(t + 3, 1, True)
            return carry

        lax.fori_loop(0, (nblk - 3) // 2, body, 0)
        drain(nblk - 3, 0)
        fire(nblk - 1, 0, True)
        drain(nblk - 2, 1)
        drain(nblk - 1, 0)
        for b in range(nb):
            pltpu.make_async_copy(
                g_hbm.at[pl.ds(0, ch)], rows.at[1, b], sem_s[1]).wait()
        for b in range(nb):
            pltpu.make_async_copy(
                g_hbm.at[pl.ds(0, ch)], rows.at[0, b], sem_s[0]).wait()
        plsc.subcore_barrier()
        pltpu.sync_copy(acc.at[pl.ds(r0, rpt)], out_hbm.at[c, pl.ds(r0, rpt)])

    return agg_kernel


def _dinv_from_degp(degp):
    deg = degp[0, :, 0:1] + degp[1, :, 0:1] + 1.0
    return lax.rsqrt(deg)


def _tca_body(x_ref, degp_ref, w_ref, out_ref):
    dinv = _dinv_from_degp(degp_ref[...])
    out_ref[...] = dinv * jnp.dot(
        x_ref[...], w_ref[...], preferred_element_type=jnp.float32
    )


def _tcb_body(sp_ref, g_ref, degp_ref, b_ref, w_ref, out_ref):
    dinv = _dinv_from_degp(degp_ref[...])
    g = g_ref[...]
    pre = dinv * (sp_ref[0] + sp_ref[1] + g) + b_ref[...]
    a = jnp.where(pre > 0, pre, jnp.exp(pre) - 1.0)
    out_ref[...] = dinv * jnp.dot(
        a, w_ref[...], preferred_element_type=jnp.float32
    )


def _tcc_body(sp_ref, g_ref, degp_ref, b_ref, wfc_ref, bfc_ref, out_ref):
    dinv = _dinv_from_degp(degp_ref[...])
    g = g_ref[...]
    pre = dinv * (sp_ref[0] + sp_ref[1] + g) + b_ref[...]
    a = jnp.where(pre > 0, pre, jnp.exp(pre) - 1.0)
    logits = jnp.dot(a, wfc_ref[...], preferred_element_type=jnp.float32)
    logits = logits + bfc_ref[...]
    m = jnp.max(logits, axis=-1, keepdims=True)
    lse = jnp.log(jnp.sum(jnp.exp(logits - m), axis=-1, keepdims=True))
    out_ref[...] = logits - m - lse


def kernel(x, edge_index, W1, b1, W2, b2, W3, b3, Wfc, bfc):
    n, f_in = x.shape
    h = W1.shape[1]
    c_out = Wfc.shape[1]
    e = edge_index.shape[1]
    ch = 80

    src = edge_index[0]
    dst = edge_index[1]
    np_ = ((n + NS * 8 - 1) // (NS * 8)) * (NS * 8)
    zeros_h = jnp.zeros((np_, h), jnp.float32)

    deg_k = _make_deg_kernel(np_, h, e, ch)
    agg_k = _make_agg_kernel(np_, h, e, ch)

    rb = 1000
    grid = (n // rb,)
    full = lambda shape: pl.BlockSpec(shape, lambda i: (0,) * len(shape))
    rows128 = pl.BlockSpec((rb, h), lambda i: (i, 0))
    degp_spec = pl.BlockSpec((NC, rb, h), lambda i: (0, i, 0))
    sp_spec = pl.BlockSpec((NC, rb, h), lambda i: (0, i, 0))

    degp = deg_k(dst, zeros_h)

    tca = pl.pallas_call(
        _tca_body,
        grid=grid,
        in_specs=[
            pl.BlockSpec((rb, f_in), lambda i: (i, 0)),
            degp_spec,
            full((f_in, h)),
        ],
        out_specs=rows128,
        out_shape=jax.ShapeDtypeStruct((n, h), jnp.float32),
    )
    g1 = tca(x, degp, W1)

    tcb = pl.pallas_call(
        _tcb_body,
        grid=grid,
        in_specs=[sp_spec, rows128, degp_spec, full((1, h)), full((h, h))],
        out_specs=rows128,
        out_shape=jax.ShapeDtypeStruct((n, h), jnp.float32),
    )
    tcc = pl.pallas_call(
        _tcc_body,
        grid=grid,
        in_specs=[
            sp_spec,
            rows128,
            degp_spec,
            full((1, h)),
            full((h, c_out)),
            full((1, c_out)),
        ],
        out_specs=pl.BlockSpec((rb, c_out), lambda i: (i, 0)),
        out_shape=jax.ShapeDtypeStruct((n, c_out), jnp.float32),
    )

    sp1 = agg_k(src, dst, g1, zeros_h)
    g2 = tcb(sp1, g1, degp, b1.reshape(1, h), W2)
    sp2 = agg_k(src, dst, g2, zeros_h)
    g3 = tcb(sp2, g2, degp, b2.reshape(1, h), W3)
    sp3 = agg_k(src, dst, g3, zeros_h)
    return tcc(sp3, g3, degp, b3.reshape(1, h), Wfc, bfc.reshape(1, c_out))


# speedup vs baseline: 23.0457x; 1.0669x over previous
import functools

import jax
import jax.numpy as jnp
from jax import lax
from jax.experimental import pallas as pl
from jax.experimental.pallas import tpu as pltpu
from jax.experimental.pallas import tpu_sc as plsc

NC = 2
NS = 16
NW = NC * NS

def _make_deg_kernel(np_, f, e, ch):
    epw = e // NW
    nch = epw // ch
    rpt = np_ // NS
    lanes = 16
    mesh = plsc.VectorSubcoreMesh(core_axis_name="c", subcore_axis_name="s")

    @functools.partial(
        pl.kernel,
        mesh=mesh,
        out_type=jax.ShapeDtypeStruct((NC, np_, f), jnp.float32),
        scratch_types=[
            pltpu.VMEM((ch,), jnp.int32),
            pltpu.VMEM((ch, f), jnp.float32),
            pltpu.VMEM_SHARED((np_, f), jnp.float32),
        ],
    )
    def deg_kernel(dst_hbm, z_hbm, out_hbm, didx, ones_v, acc):
        c = lax.axis_index("c")
        s = lax.axis_index("s")
        wid = s * NC + c
        r0 = s * rpt

        def fill(j, carry):
            r = j // (f // lanes)
            col = (j % (f // lanes)) * lanes
            ones_v[r, pl.ds(col, lanes)] = jnp.full((lanes,), 1.0, jnp.float32)
            return carry

        lax.fori_loop(0, ch * (f // lanes), fill, 0)
        pltpu.sync_copy(z_hbm.at[pl.ds(r0, rpt)], acc.at[pl.ds(r0, rpt)])
        plsc.subcore_barrier()
        base = wid * epw

        def body(j, carry):
            off = pl.multiple_of(base + j * ch, 8)
            pltpu.sync_copy(dst_hbm.at[pl.ds(off, ch)], didx)
            pltpu.sync_copy(ones_v, acc.at[didx], add=True)
            return carry

        lax.fori_loop(0, nch, body, 0)
        plsc.subcore_barrier()
        pltpu.sync_copy(acc.at[pl.ds(r0, rpt)], out_hbm.at[c, pl.ds(r0, rpt)])

    return deg_kernel


def _make_agg_kernel(np_, f, e, ch):
    epw = e // NW
    nch = epw // ch
    rpt = np_ // NS
    mesh = plsc.VectorSubcoreMesh(core_axis_name="c", subcore_axis_name="s")

    nb = 1
    nblk = nch // nb
    assert nch % nb == 0 and nblk >= 3 and nblk % 2 == 1

    @functools.partial(
        pl.kernel,
        mesh=mesh,
        out_type=jax.ShapeDtypeStruct((NC, np_, f), jnp.float32),
        scratch_types=[
            pltpu.VMEM((epw,), jnp.int32),
            pltpu.VMEM((nch, ch), jnp.int32),
            pltpu.VMEM((2, nb, ch, f), jnp.float32),
            pltpu.VMEM_SHARED((np_, f), jnp.float32),
            pltpu.SemaphoreType.DMA,
            pltpu.SemaphoreType.DMA,
            pltpu.SemaphoreType.DMA,
            pltpu.SemaphoreType.DMA,
            pltpu.SemaphoreType.DMA,
            pltpu.SemaphoreType.DMA,
        ],
    )
    def agg_kernel(src_hbm, dst_hbm, g_hbm, z_hbm, out_hbm, sidx_all, didx_all,
                   rows, acc, si0, si1, sg0, sg1, ss0, ss1):
        c = lax.axis_index("c")
        s = lax.axis_index("s")
        wid = s * NC + c
        r0 = s * rpt
        base = wid * epw
        sem_i = (si0, si1)
        sem_g = (sg0, sg1)
        sem_s = (ss0, ss1)

        sidx_cp = pltpu.async_copy(
            src_hbm.at[pl.ds(pl.multiple_of(base, 8), epw)], sidx_all, si0)
        pltpu.sync_copy(z_hbm.at[pl.ds(r0, rpt)], acc.at[pl.ds(r0, rpt)])
        plsc.subcore_barrier()
        sidx_cp.wait()

        def fire(t, st, drain_s):
            if drain_s:
                for b in range(nb):
                    pltpu.make_async_copy(
                        g_hbm.at[pl.ds(0, ch)], rows.at[st, b], sem_s[st]).wait()
            for b in range(nb):
                j = t * nb + b
                off = pl.multiple_of(base + j * ch, 8)
                pltpu.async_copy(dst_hbm.at[pl.ds(off, ch)], didx_all.at[j],
                                 sem_i[st])
                goff = pl.multiple_of(j * ch, 8)
                pltpu.async_copy(g_hbm.at[sidx_all.at[pl.ds(goff, ch)]],
                                 rows.at[st, b], sem_g[st])

        def drain(t, st):
            for b in range(nb):
                pltpu.make_async_copy(
                    g_hbm.at[pl.ds(0, ch)], rows.at[st, b], sem_g[st]).wait()
            for b in range(nb):
                j = t * nb + b
                pltpu.make_async_copy(
                    dst_hbm.at[pl.ds(0, ch)], didx_all.at[j], sem_i[st]).wait()
                pltpu.async_copy(rows.at[st, b], acc.at[didx_all.at[j]],
                                 sem_s[st], add=True)

        fire(0, 0, False)
        fire(1, 1, False)

        def body(i, carry):
            t = 2 * i
            drain(t, 0)
            fire(t + 2, 0, True)
            drain(t + 1, 1)
            fire(t + 3, 1, True)
            return carry

        lax.fori_loop(0, (nblk - 3) // 2, body, 0)
        drain(nblk - 3, 0)
        fire(nblk - 1, 0, True)
        drain(nblk - 2, 1)
        drain(nblk - 1, 0)
        for b in range(nb):
            pltpu.make_async_copy(
                g_hbm.at[pl.ds(0, ch)], rows.at[1, b], sem_s[1]).wait()
        for b in range(nb):
            pltpu.make_async_copy(
                g_hbm.at[pl.ds(0, ch)], rows.at[0, b], sem_s[0]).wait()
        plsc.subcore_barrier()
        pltpu.sync_copy(acc.at[pl.ds(r0, rpt)], out_hbm.at[c, pl.ds(r0, rpt)])

    return agg_kernel


def _tca_body(x_ref, degp_ref, w_ref, out_ref, dinv_ref):
    deg = degp_ref[0, :, 0:1] + degp_ref[1, :, 0:1] + 1.0
    dinv = lax.rsqrt(deg)
    dinv_ref[...] = dinv
    out_ref[...] = dinv * jnp.dot(
        x_ref[...], w_ref[...], preferred_element_type=jnp.float32
    )


def _tcb_body(sp_ref, g_ref, dinv_ref, b_ref, w_ref, out_ref):
    dinv = dinv_ref[...]
    g = g_ref[...]
    pre = dinv * (sp_ref[0] + sp_ref[1] + g) + b_ref[...]
    a = jnp.where(pre > 0, pre, jnp.exp(pre) - 1.0)
    out_ref[...] = dinv * jnp.dot(
        a, w_ref[...], preferred_element_type=jnp.float32
    )


def _tcc_body(sp_ref, g_ref, dinv_ref, b_ref, wfc_ref, bfc_ref, out_ref):
    dinv = dinv_ref[...]
    g = g_ref[...]
    pre = dinv * (sp_ref[0] + sp_ref[1] + g) + b_ref[...]
    a = jnp.where(pre > 0, pre, jnp.exp(pre) - 1.0)
    logits = jnp.dot(a, wfc_ref[...], preferred_element_type=jnp.float32)
    logits = logits + bfc_ref[...]
    m = jnp.max(logits, axis=-1, keepdims=True)
    lse = jnp.log(jnp.sum(jnp.exp(logits - m), axis=-1, keepdims=True))
    out_ref[...] = logits - m - lse


def kernel(x, edge_index, W1, b1, W2, b2, W3, b3, Wfc, bfc):
    n, f_in = x.shape
    h = W1.shape[1]
    c_out = Wfc.shape[1]
    e = edge_index.shape[1]
    ch = 80

    src = edge_index[0]
    dst = edge_index[1]
    np_ = ((n + NS * 8 - 1) // (NS * 8)) * (NS * 8)
    zeros_h = jnp.zeros((np_, h), jnp.float32)

    deg_k = _make_deg_kernel(np_, h, e, ch)
    agg_k = _make_agg_kernel(np_, h, e, ch)

    rb = 1000
    grid = (n // rb,)
    full = lambda shape: pl.BlockSpec(shape, lambda i: (0,) * len(shape))
    rows128 = pl.BlockSpec((rb, h), lambda i: (i, 0))
    degp_spec = pl.BlockSpec((NC, rb, h), lambda i: (0, i, 0))
    sp_spec = pl.BlockSpec((NC, rb, h), lambda i: (0, i, 0))
    dinv_spec = pl.BlockSpec((rb, 1), lambda i: (i, 0))

    degp = deg_k(dst, zeros_h)

    tca = pl.pallas_call(
        _tca_body,
        grid=grid,
        in_specs=[
            pl.BlockSpec((rb, f_in), lambda i: (i, 0)),
            degp_spec,
            full((f_in, h)),
        ],
        out_specs=[rows128, dinv_spec],
        out_shape=[
            jax.ShapeDtypeStruct((n, h), jnp.float32),
            jax.ShapeDtypeStruct((n, 1), jnp.float32),
        ],
    )
    g1, dinv = tca(x, degp, W1)

    tcb = pl.pallas_call(
        _tcb_body,
        grid=grid,
        in_specs=[sp_spec, rows128, dinv_spec, full((1, h)), full((h, h))],
        out_specs=rows128,
        out_shape=jax.ShapeDtypeStruct((n, h), jnp.float32),
    )
    tcc = pl.pallas_call(
        _tcc_body,
        grid=grid,
        in_specs=[
            sp_spec,
            rows128,
            dinv_spec,
            full((1, h)),
            full((h, c_out)),
            full((1, c_out)),
        ],
        out_specs=pl.BlockSpec((rb, c_out), lambda i: (i, 0)),
        out_shape=jax.ShapeDtypeStruct((n, c_out), jnp.float32),
    )

    sp1 = agg_k(src, dst, g1, zeros_h)
    g2 = tcb(sp1, g1, dinv, b1.reshape(1, h), W2)
    sp2 = agg_k(src, dst, g2, zeros_h)
    g3 = tcb(sp2, g2, dinv, b2.reshape(1, h), W3)
    sp3 = agg_k(src, dst, g3, zeros_h)
    return tcc(sp3, g3, dinv, b3.reshape(1, h), Wfc, bfc.reshape(1, c_out))


# speedup vs baseline: 23.0723x; 1.0012x over previous
import functools

import jax
import jax.numpy as jnp
from jax import lax
from jax.experimental import pallas as pl
from jax.experimental.pallas import tpu as pltpu
from jax.experimental.pallas import tpu_sc as plsc

NC = 2
NS = 16
NW = NC * NS

DEG_W = 32


def _make_deg_kernel(np_, f, e, ch):
    epw = e // NW
    nch = epw // ch
    rpt = np_ // NS
    lanes = 16
    mesh = plsc.VectorSubcoreMesh(core_axis_name="c", subcore_axis_name="s")

    @functools.partial(
        pl.kernel,
        mesh=mesh,
        out_type=jax.ShapeDtypeStruct((NC, np_, f), jnp.float32),
        scratch_types=[
            pltpu.VMEM((ch,), jnp.int32),
            pltpu.VMEM((ch, f), jnp.float32),
            pltpu.VMEM_SHARED((np_, f), jnp.float32),
        ],
    )
    def deg_kernel(dst_hbm, z_hbm, out_hbm, didx, ones_v, acc):
        c = lax.axis_index("c")
        s = lax.axis_index("s")
        wid = s * NC + c
        r0 = s * rpt

        def fill(j, carry):
            r = j // (f // lanes)
            col = (j % (f // lanes)) * lanes
            ones_v[r, pl.ds(col, lanes)] = jnp.full((lanes,), 1.0, jnp.float32)
            return carry

        lax.fori_loop(0, ch * (f // lanes), fill, 0)
        pltpu.sync_copy(z_hbm.at[pl.ds(r0, rpt)], acc.at[pl.ds(r0, rpt)])
        plsc.subcore_barrier()
        base = wid * epw

        def body(j, carry):
            off = pl.multiple_of(base + j * ch, 8)
            pltpu.sync_copy(dst_hbm.at[pl.ds(off, ch)], didx)
            pltpu.sync_copy(ones_v, acc.at[didx], add=True)
            return carry

        lax.fori_loop(0, nch, body, 0)
        plsc.subcore_barrier()
        pltpu.sync_copy(acc.at[pl.ds(r0, rpt)], out_hbm.at[c, pl.ds(r0, rpt)])

    return deg_kernel


def _make_agg_kernel(np_, f, e, ch):
    epw = e // NW
    nch = epw // ch
    rpt = np_ // NS
    mesh = plsc.VectorSubcoreMesh(core_axis_name="c", subcore_axis_name="s")

    nb = 1
    nblk = nch // nb
    assert nch % nb == 0 and nblk >= 3 and nblk % 2 == 1

    @functools.partial(
        pl.kernel,
        mesh=mesh,
        out_type=jax.ShapeDtypeStruct((NC, np_, f), jnp.float32),
        scratch_types=[
            pltpu.VMEM((epw,), jnp.int32),
            pltpu.VMEM((nch, ch), jnp.int32),
            pltpu.VMEM((2, nb, ch, f), jnp.float32),
            pltpu.VMEM_SHARED((np_, f), jnp.float32),
            pltpu.SemaphoreType.DMA,
            pltpu.SemaphoreType.DMA,
            pltpu.SemaphoreType.DMA,
            pltpu.SemaphoreType.DMA,
            pltpu.SemaphoreType.DMA,
            pltpu.SemaphoreType.DMA,
        ],
    )
    def agg_kernel(src_hbm, dst_hbm, g_hbm, z_hbm, out_hbm, sidx_all, didx_all,
                   rows, acc, si0, si1, sg0, sg1, ss0, ss1):
        c = lax.axis_index("c")
        s = lax.axis_index("s")
        wid = s * NC + c
        r0 = s * rpt
        base = wid * epw
        sem_i = (si0, si1)
        sem_g = (sg0, sg1)
        sem_s = (ss0, ss1)

        sidx_cp = pltpu.async_copy(
            src_hbm.at[pl.ds(pl.multiple_of(base, 8), epw)], sidx_all, si0)
        pltpu.sync_copy(z_hbm.at[pl.ds(r0, rpt)], acc.at[pl.ds(r0, rpt)])
        plsc.subcore_barrier()
        sidx_cp.wait()

        def fire(t, st, drain_s):
            if drain_s:
                for b in range(nb):
                    pltpu.make_async_copy(
                        g_hbm.at[pl.ds(0, ch)], rows.at[st, b], sem_s[st]).wait()
            for b in range(nb):
                j = t * nb + b
                off = pl.multiple_of(base + j * ch, 8)
                pltpu.async_copy(dst_hbm.at[pl.ds(off, ch)], didx_all.at[j],
                                 sem_i[st])
                goff = pl.multiple_of(j * ch, 8)
                pltpu.async_copy(g_hbm.at[sidx_all.at[pl.ds(goff, ch)]],
                                 rows.at[st, b], sem_g[st])

        def drain(t, st):
            for b in range(nb):
                pltpu.make_async_copy(
                    g_hbm.at[pl.ds(0, ch)], rows.at[st, b], sem_g[st]).wait()
            for b in range(nb):
                j = t * nb + b
                pltpu.make_async_copy(
                    dst_hbm.at[pl.ds(0, ch)], didx_all.at[j], sem_i[st]).wait()
                pltpu.async_copy(rows.at[st, b], acc.at[didx_all.at[j]],
                                 sem_s[st], add=True)

        fire(0, 0, False)
        fire(1, 1, False)

        def body(i, carry):
            t = 2 * i
            drain(t, 0)
            fire(t + 2, 0, True)
            drain(t + 1, 1)
            fire(t + 3, 1, True)
            return carry

        lax.fori_loop(0, (nblk - 3) // 2, body, 0)
        drain(nblk - 3, 0)
        fire(nblk - 1, 0, True)
        drain(nblk - 2, 1)
        drain(nblk - 1, 0)
        for b in range(nb):
            pltpu.make_async_copy(
                g_hbm.at[pl.ds(0, ch)], rows.at[1, b], sem_s[1]).wait()
        for b in range(nb):
            pltpu.make_async_copy(
                g_hbm.at[pl.ds(0, ch)], rows.at[0, b], sem_s[0]).wait()
        plsc.subcore_barrier()
        pltpu.sync_copy(acc.at[pl.ds(r0, rpt)], out_hbm.at[c, pl.ds(r0, rpt)])

    return agg_kernel


def _tca_body(x_ref, degp_ref, w_ref, out_ref, dinv_ref):
    deg = degp_ref[0, :, 0:1] + degp_ref[1, :, 0:1] + 1.0
    dinv = lax.rsqrt(deg)
    dinv_ref[...] = dinv
    out_ref[...] = dinv * jnp.dot(
        x_ref[...], w_ref[...], preferred_element_type=jnp.float32
    )


def _tcb_body(sp_ref, g_ref, dinv_ref, b_ref, w_ref, out_ref):
    dinv = dinv_ref[...]
    g = g_ref[...]
    pre = dinv * (sp_ref[0] + sp_ref[1] + g) + b_ref[...]
    a = jnp.where(pre > 0, pre, jnp.exp(pre) - 1.0)
    out_ref[...] = dinv * jnp.dot(
        a, w_ref[...], preferred_element_type=jnp.float32
    )


def _tcc_body(sp_ref, g_ref, dinv_ref, b_ref, wfc_ref, bfc_ref, out_ref):
    dinv = dinv_ref[...]
    g = g_ref[...]
    pre = dinv * (sp_ref[0] + sp_ref[1] + g) + b_ref[...]
    a = jnp.where(pre > 0, pre, jnp.exp(pre) - 1.0)
    logits = jnp.dot(a, wfc_ref[...], preferred_element_type=jnp.float32)
    logits = logits + bfc_ref[...]
    m = jnp.max(logits, axis=-1, keepdims=True)
    lse = jnp.log(jnp.sum(jnp.exp(logits - m), axis=-1, keepdims=True))
    out_ref[...] = logits - m - lse


def kernel(x, edge_index, W1, b1, W2, b2, W3, b3, Wfc, bfc):
    n, f_in = x.shape
    h = W1.shape[1]
    c_out = Wfc.shape[1]
    e = edge_index.shape[1]
    ch = 80

    src = edge_index[0]
    dst = edge_index[1]
    np_ = ((n + NS * 8 - 1) // (NS * 8)) * (NS * 8)
    zeros_h = jnp.zeros((np_, h), jnp.float32)
    zeros_d = jnp.zeros((np_, DEG_W), jnp.float32)

    deg_k = _make_deg_kernel(np_, DEG_W, e, ch)
    agg_k = _make_agg_kernel(np_, h, e, ch)

    rb = 1000
    grid = (n // rb,)
    full = lambda shape: pl.BlockSpec(shape, lambda i: (0,) * len(shape))
    rows128 = pl.BlockSpec((rb, h), lambda i: (i, 0))
    degp_spec = pl.BlockSpec((NC, rb, DEG_W), lambda i: (0, i, 0))
    sp_spec = pl.BlockSpec((NC, rb, h), lambda i: (0, i, 0))
    dinv_spec = pl.BlockSpec((rb, 1), lambda i: (i, 0))

    degp = deg_k(dst, zeros_d)

    tca = pl.pallas_call(
        _tca_body,
        grid=grid,
        in_specs=[
            pl.BlockSpec((rb, f_in), lambda i: (i, 0)),
            degp_spec,
            full((f_in, h)),
        ],
        out_specs=[rows128, dinv_spec],
        out_shape=[
            jax.ShapeDtypeStruct((n, h), jnp.float32),
            jax.ShapeDtypeStruct((n, 1), jnp.float32),
        ],
    )
    g1, dinv = tca(x, degp, W1)

    tcb = pl.pallas_call(
        _tcb_body,
        grid=grid,
        in_specs=[sp_spec, rows128, dinv_spec, full((1, h)), full((h, h))],
        out_specs=rows128,
        out_shape=jax.ShapeDtypeStruct((n, h), jnp.float32),
    )
    tcc = pl.pallas_call(
        _tcc_body,
        grid=grid,
        in_specs=[
            sp_spec,
            rows128,
            dinv_spec,
            full((1, h)),
            full((h, c_out)),
            full((1, c_out)),
        ],
        out_specs=pl.BlockSpec((rb, c_out), lambda i: (i, 0)),
        out_shape=jax.ShapeDtypeStruct((n, c_out), jnp.float32),
    )

    sp1 = agg_k(src, dst, g1, zeros_h)
    g2 = tcb(sp1, g1, dinv, b1.reshape(1, h), W2)
    sp2 = agg_k(src, dst, g2, zeros_h)
    g3 = tcb(sp2, g2, dinv, b2.reshape(1, h), W3)
    sp3 = agg_k(src, dst, g3, zeros_h)
    return tcc(sp3, g3, dinv, b3.reshape(1, h), Wfc, bfc.reshape(1, c_out))
